# baseline (speedup 1.0000x reference)
import jax
import jax.numpy as jnp
from jax.experimental import pallas as pl
from jax.experimental.pallas import tpu as pltpu

_BLK = 512


def _body(idx_ref, w_ref, x_ref, o_ref):
    scale = idx_ref[0, 0, :].astype(jnp.float32)[:, None]
    o_ref[...] = x_ref[...] + scale * w_ref[...]


def kernel(x, indices, embs_weight):
    B, S, D = x.shape
    n = (B * S) // _BLK
    x2 = x.reshape(n * _BLK, D)
    idx3 = indices.reshape(n, 1, _BLK)
    out = pl.pallas_call(
        _body,
        grid=(n,),
        in_specs=[
            pl.BlockSpec((1, 1, _BLK), lambda i: (i, 0, 0)),
            pl.BlockSpec((1, D), lambda i: (0, 0)),
            pl.BlockSpec((_BLK, D), lambda i: (i, 0)),
        ],
        out_specs=pl.BlockSpec((_BLK, D), lambda i: (i, 0)),
        out_shape=jax.ShapeDtypeStruct((n * _BLK, D), x.dtype),
        compiler_params=pltpu.CompilerParams(
            dimension_semantics=("parallel",),
        ),
    )(idx3, embs_weight, x2)
    return out.reshape(B, S, D)


# speedup vs baseline: 1.1123x; 1.1123x over previous
import jax
import jax.numpy as jnp
from jax.experimental import pallas as pl
from jax.experimental.pallas import tpu as pltpu

_BLK = 2048


def _body(idx_ref, w_ref, x_ref, o_ref):
    scale = idx_ref[0, 0, :].astype(jnp.float32)[:, None]
    o_ref[...] = x_ref[...] + scale * w_ref[...]


def kernel(x, indices, embs_weight):
    B, S, D = x.shape
    n = (B * S) // _BLK
    x2 = x.reshape(n * _BLK, D)
    idx3 = indices.reshape(n, 1, _BLK)
    out = pl.pallas_call(
        _body,
        grid=(n,),
        in_specs=[
            pl.BlockSpec((1, 1, _BLK), lambda i: (i, 0, 0)),
            pl.BlockSpec((1, D), lambda i: (0, 0)),
            pl.BlockSpec((_BLK, D), lambda i: (i, 0)),
        ],
        out_specs=pl.BlockSpec((_BLK, D), lambda i: (i, 0)),
        out_shape=jax.ShapeDtypeStruct((n * _BLK, D), x.dtype),
        compiler_params=pltpu.CompilerParams(
            dimension_semantics=("parallel",),
        ),
    )(idx3, embs_weight, x2)
    return out.reshape(B, S, D)
